# baseline (speedup 1.0000x reference)
import jax
import jax.numpy as jnp
from jax import lax
from jax.experimental import pallas as pl
from jax.experimental.pallas import tpu as pltpu
from jax.experimental.pallas import tpu_sc as plsc

NC = 2
NS = 16
NW = NC * NS

B_DIM = 16384
T_DIM = 200
TBLK = T_DIM // 8
BTIL = B_DIM // 128
N_PAIRS = TBLK * BTIL
PAIRS_W = N_PAIRS // NW
CHUNK_PAIRS = 4
CHUNK = CHUNK_PAIRS * 1024
N_CHUNKS = PAIRS_W // CHUNK_PAIRS
NB = 2

TBL_SLICE = 62592
TBL_BOUNCE = 15648
TBL_PAD = TBL_SLICE * NS


def _gather_body(idx_hbm, table_hbm, out_hbm,
                 idx0, idx1, rid0, rid1, out0, out1, tbl_sh, tbl_b,
                 l0, l1, g0, g1, o0, o1, tsem):
    cid = lax.axis_index("c")
    sid = lax.axis_index("s")
    wid = sid * NC + cid
    pair_base = wid * PAIRS_W

    idx_v = [idx0, idx1]
    rid_v = [rid0, rid1]
    out_v = [out0, out1]
    lsem = [l0, l1]
    gsem = [g0, g1]
    osem = [o0, o1]

    def fire_load(c, b):
        pltpu.async_copy(
            idx_hbm.at[pl.ds((pair_base + c * CHUNK_PAIRS) * 1024, CHUNK)],
            idx_v[b], lsem[b])

    def drain_load(b):
        pltpu.make_async_copy(idx_hbm.at[pl.ds(0, CHUNK)], idx_v[b],
                              lsem[b]).wait()

    def reorder(b):
        for s in range(8):
            for j in range(CHUNK_PAIRS):
                for v in range(8):
                    src = j * 1024 + s * 128 + v * 16
                    dst = s * CHUNK_PAIRS * 128 + j * 128 + v * 16
                    rid_v[b][pl.ds(dst, 16)] = idx_v[b][pl.ds(src, 16)]

    def fire_gather(b):
        pltpu.async_copy(tbl_sh.at[rid_v[b]], out_v[b], gsem[b])

    def drain_gather(b):
        pltpu.make_async_copy(table_hbm.at[pl.ds(0, CHUNK)], out_v[b],
                              gsem[b]).wait()

    def fire_stores(c, b):
        pair0 = pair_base + c * CHUNK_PAIRS
        t_blk = pair0 // BTIL
        b_off = (pair0 % BTIL) * 128
        run = CHUNK_PAIRS * 128
        for s in range(8):
            pltpu.async_copy(
                out_v[b].at[pl.ds(s * run, run)],
                out_hbm.at[pl.ds((t_blk * 8 + s) * B_DIM + b_off, run)],
                osem[b])

    def drain_stores(b):
        pltpu.make_async_copy(table_hbm.at[pl.ds(0, CHUNK)], out_v[b],
                              osem[b]).wait()

    fire_load(0, 0)
    fire_load(1, 1)

    for r in range(TBL_SLICE // TBL_BOUNCE):
        t_off = sid * TBL_SLICE + r * TBL_BOUNCE
        pltpu.async_copy(
            table_hbm.at[pl.ds(t_off, TBL_BOUNCE)], tbl_b, tsem).wait()
        pltpu.async_copy(
            tbl_b, tbl_sh.at[pl.ds(t_off, TBL_BOUNCE)], tsem).wait()
    plsc.subcore_barrier()

    def step(i, _):
        for b in (0, 1):
            drain_load(b)
            reorder(b)

            @pl.when(i > 0)
            def _():
                drain_stores(b)

            fire_gather(b)
        for b in (0, 1):
            c = 2 * i + b
            drain_gather(b)
            fire_stores(c, b)

            @pl.when(c + 2 < N_CHUNKS)
            def _():
                fire_load(c + 2, b)
        return ()

    lax.fori_loop(0, (N_CHUNKS - 1) // NB, step, (), unroll=False)

    drain_load(0)
    reorder(0)
    drain_stores(0)
    fire_gather(0)
    drain_gather(0)
    fire_stores(N_CHUNKS - 1, 0)
    drain_stores(1)
    drain_stores(0)


def kernel(item_ids, table):
    n_total = item_ids.shape[0] * item_ids.shape[1]
    flat_idx = (
        item_ids.T.reshape(TBLK, 8, BTIL, 128)
        .transpose(0, 2, 1, 3)
        .reshape(n_total)
    )
    flat_table = jnp.pad(
        table, ((0, TBL_PAD - table.shape[0]), (0, 0))).reshape(TBL_PAD)

    mesh = plsc.VectorSubcoreMesh(core_axis_name="c", subcore_axis_name="s")
    flat_out = pl.kernel(
        _gather_body,
        out_type=jax.ShapeDtypeStruct((n_total,), jnp.float32),
        mesh=mesh,
        scratch_types=[
            pltpu.VMEM((CHUNK,), jnp.int32),
            pltpu.VMEM((CHUNK,), jnp.int32),
            pltpu.VMEM((CHUNK,), jnp.int32),
            pltpu.VMEM((CHUNK,), jnp.int32),
            pltpu.VMEM((CHUNK,), jnp.float32),
            pltpu.VMEM((CHUNK,), jnp.float32),
            pltpu.MemorySpace.VMEM_SHARED((TBL_PAD,), jnp.float32),
            pltpu.VMEM((TBL_BOUNCE,), jnp.float32),
            pltpu.SemaphoreType.DMA,
            pltpu.SemaphoreType.DMA,
            pltpu.SemaphoreType.DMA,
            pltpu.SemaphoreType.DMA,
            pltpu.SemaphoreType.DMA,
            pltpu.SemaphoreType.DMA,
            pltpu.SemaphoreType.DMA,
        ],
    )(flat_idx, flat_table)
    return flat_out.reshape(T_DIM, B_DIM, 1).transpose(1, 0, 2)


# speedup vs baseline: 1.1429x; 1.1429x over previous
import jax
import jax.numpy as jnp
from jax import lax
from jax.experimental import pallas as pl
from jax.experimental.pallas import tpu as pltpu
from jax.experimental.pallas import tpu_sc as plsc

NC = 2
NS = 16
NW = NC * NS

B_DIM = 16384
T_DIM = 200
TBLK = T_DIM // 8
BTIL = B_DIM // 128
N_PAIRS = TBLK * BTIL
PAIRS_W = N_PAIRS // NW
CHUNK_PAIRS = 4
CHUNK = CHUNK_PAIRS * 1024
N_CHUNKS = PAIRS_W // CHUNK_PAIRS
NB = 3

TBL_SLICE = 62592
TBL_BOUNCE = 15648
TBL_PAD = TBL_SLICE * NS


def _gather_body(idx_hbm, table_hbm, out_hbm,
                 idx0, idx1, idx2, rid0, rid1, rid2, out0, out1, out2,
                 tbl_sh, tbl_b,
                 l0, l1, l2, g0, g1, g2, o0, o1, o2, tsem):
    cid = lax.axis_index("c")
    sid = lax.axis_index("s")
    wid = sid * NC + cid
    pair_base = wid * PAIRS_W

    idx_v = [idx0, idx1, idx2]
    rid_v = [rid0, rid1, rid2]
    out_v = [out0, out1, out2]
    lsem = [l0, l1, l2]
    gsem = [g0, g1, g2]
    osem = [o0, o1, o2]

    def fire_load(c, b):
        pltpu.async_copy(
            idx_hbm.at[pl.ds((pair_base + c * CHUNK_PAIRS) * 1024, CHUNK)],
            idx_v[b], lsem[b])

    def drain_load(b):
        pltpu.make_async_copy(idx_hbm.at[pl.ds(0, CHUNK)], idx_v[b],
                              lsem[b]).wait()

    def reorder(b):
        for s in range(8):
            for j in range(CHUNK_PAIRS):
                for v in range(8):
                    src = j * 1024 + s * 128 + v * 16
                    dst = s * CHUNK_PAIRS * 128 + j * 128 + v * 16
                    rid_v[b][pl.ds(dst, 16)] = idx_v[b][pl.ds(src, 16)]

    def fire_gather(b):
        pltpu.async_copy(tbl_sh.at[rid_v[b]], out_v[b], gsem[b])

    def drain_gather(b):
        pltpu.make_async_copy(table_hbm.at[pl.ds(0, CHUNK)], out_v[b],
                              gsem[b]).wait()

    def fire_stores(c, b):
        pair0 = pair_base + c * CHUNK_PAIRS
        t_blk = pair0 // BTIL
        b_off = (pair0 % BTIL) * 128
        run = CHUNK_PAIRS * 128
        for s in range(8):
            pltpu.async_copy(
                out_v[b].at[pl.ds(s * run, run)],
                out_hbm.at[pl.ds((t_blk * 8 + s) * B_DIM + b_off, run)],
                osem[b])

    def drain_stores(b):
        pltpu.make_async_copy(table_hbm.at[pl.ds(0, CHUNK)], out_v[b],
                              osem[b]).wait()

    fire_load(0, 0)
    fire_load(1, 1)
    fire_load(2, 2)

    for r in range(TBL_SLICE // TBL_BOUNCE):
        t_off = sid * TBL_SLICE + r * TBL_BOUNCE
        pltpu.async_copy(
            table_hbm.at[pl.ds(t_off, TBL_BOUNCE)], tbl_b, tsem).wait()
        pltpu.async_copy(
            tbl_b, tbl_sh.at[pl.ds(t_off, TBL_BOUNCE)], tsem).wait()
    plsc.subcore_barrier()

    def step(i, _):
        for b in (0, 1, 2):
            drain_load(b)
            reorder(b)

            @pl.when(i > 0)
            def _():
                drain_stores(b)

            fire_gather(b)
        for b in (0, 1, 2):
            c = NB * i + b
            drain_gather(b)
            fire_stores(c, b)

            @pl.when(c + NB < N_CHUNKS)
            def _():
                fire_load(c + NB, b)
        return ()

    lax.fori_loop(0, (N_CHUNKS - 1) // NB, step, (), unroll=False)

    drain_load(0)
    reorder(0)
    drain_stores(0)
    fire_gather(0)
    drain_gather(0)
    fire_stores(N_CHUNKS - 1, 0)
    drain_stores(1)
    drain_stores(2)
    drain_stores(0)


def kernel(item_ids, table):
    n_total = item_ids.shape[0] * item_ids.shape[1]
    flat_idx = (
        item_ids.T.reshape(TBLK, 8, BTIL, 128)
        .transpose(0, 2, 1, 3)
        .reshape(n_total)
    )
    flat_table = jnp.pad(
        table, ((0, TBL_PAD - table.shape[0]), (0, 0))).reshape(TBL_PAD)

    mesh = plsc.VectorSubcoreMesh(core_axis_name="c", subcore_axis_name="s")
    flat_out = pl.kernel(
        _gather_body,
        out_type=jax.ShapeDtypeStruct((n_total,), jnp.float32),
        mesh=mesh,
        scratch_types=(
            [pltpu.VMEM((CHUNK,), jnp.int32)] * 6
            + [pltpu.VMEM((CHUNK,), jnp.float32)] * 3
            + [pltpu.MemorySpace.VMEM_SHARED((TBL_PAD,), jnp.float32)]
            + [pltpu.VMEM((TBL_BOUNCE,), jnp.float32)]
            + [pltpu.SemaphoreType.DMA] * 10
        ),
    )(flat_idx, flat_table)
    return flat_out.reshape(T_DIM, B_DIM, 1).transpose(1, 0, 2)


# speedup vs baseline: 1.1436x; 1.0006x over previous
import jax
import jax.numpy as jnp
from jax import lax
from jax.experimental import pallas as pl
from jax.experimental.pallas import tpu as pltpu
from jax.experimental.pallas import tpu_sc as plsc

NC = 2
NS = 16
NW = NC * NS

B_DIM = 16384
T_DIM = 200
TBLK = T_DIM // 8
BTIL = B_DIM // 128
N_PAIRS = TBLK * BTIL
PAIRS_W = N_PAIRS // NW
CHUNK_PAIRS = 4
CHUNK = CHUNK_PAIRS * 1024
N_CHUNKS = PAIRS_W // CHUNK_PAIRS
NB = 4

TBL_SLICE = 62592
TBL_BOUNCE = 15648
TBL_PAD = TBL_SLICE * NS


def _gather_body(idx_hbm, table_hbm, out_hbm,
                 idx0, idx1, idx2, idx3, rid0, rid1, rid2, rid3,
                 out0, out1, out2, out3, tbl_sh, tbl_b,
                 l0, l1, l2, l3, g0, g1, g2, g3, o0, o1, o2, o3, tsem):
    cid = lax.axis_index("c")
    sid = lax.axis_index("s")
    wid = sid * NC + cid
    pair_base = wid * PAIRS_W

    idx_v = [idx0, idx1, idx2, idx3]
    rid_v = [rid0, rid1, rid2, rid3]
    out_v = [out0, out1, out2, out3]
    lsem = [l0, l1, l2, l3]
    gsem = [g0, g1, g2, g3]
    osem = [o0, o1, o2, o3]

    def fire_load(c, b):
        pltpu.async_copy(
            idx_hbm.at[pl.ds((pair_base + c * CHUNK_PAIRS) * 1024, CHUNK)],
            idx_v[b], lsem[b])

    def drain_load(b):
        pltpu.make_async_copy(idx_hbm.at[pl.ds(0, CHUNK)], idx_v[b],
                              lsem[b]).wait()

    def reorder(b):
        for s in range(8):
            for j in range(CHUNK_PAIRS):
                for v in range(8):
                    src = j * 1024 + s * 128 + v * 16
                    dst = s * CHUNK_PAIRS * 128 + j * 128 + v * 16
                    rid_v[b][pl.ds(dst, 16)] = idx_v[b][pl.ds(src, 16)]

    def fire_gather(b):
        pltpu.async_copy(tbl_sh.at[rid_v[b]], out_v[b], gsem[b])

    def drain_gather(b):
        pltpu.make_async_copy(table_hbm.at[pl.ds(0, CHUNK)], out_v[b],
                              gsem[b]).wait()

    def fire_stores(c, b):
        pair0 = pair_base + c * CHUNK_PAIRS
        t_blk = pair0 // BTIL
        b_off = (pair0 % BTIL) * 128
        run = CHUNK_PAIRS * 128
        for s in range(8):
            pltpu.async_copy(
                out_v[b].at[pl.ds(s * run, run)],
                out_hbm.at[pl.ds((t_blk * 8 + s) * B_DIM + b_off, run)],
                osem[b])

    def drain_stores(b):
        pltpu.make_async_copy(table_hbm.at[pl.ds(0, CHUNK)], out_v[b],
                              osem[b]).wait()

    fire_load(0, 0)
    fire_load(1, 1)
    fire_load(2, 2)
    fire_load(3, 3)

    for r in range(TBL_SLICE // TBL_BOUNCE):
        t_off = sid * TBL_SLICE + r * TBL_BOUNCE
        pltpu.async_copy(
            table_hbm.at[pl.ds(t_off, TBL_BOUNCE)], tbl_b, tsem).wait()
        pltpu.async_copy(
            tbl_b, tbl_sh.at[pl.ds(t_off, TBL_BOUNCE)], tsem).wait()
    plsc.subcore_barrier()

    def step(i, _):
        for b in (0, 1, 2, 3):
            drain_load(b)
            reorder(b)

            @pl.when(i > 0)
            def _():
                drain_stores(b)

            fire_gather(b)
        for b in (0, 1, 2, 3):
            c = NB * i + b
            drain_gather(b)
            fire_stores(c, b)

            @pl.when(c + NB < N_CHUNKS)
            def _():
                fire_load(c + NB, b)
        return ()

    lax.fori_loop(0, (N_CHUNKS - 1) // NB, step, (), unroll=False)

    drain_load(0)
    reorder(0)
    drain_stores(0)
    fire_gather(0)
    drain_gather(0)
    fire_stores(N_CHUNKS - 1, 0)
    drain_stores(1)
    drain_stores(2)
    drain_stores(3)
    drain_stores(0)


def kernel(item_ids, table):
    n_total = item_ids.shape[0] * item_ids.shape[1]
    flat_idx = (
        item_ids.T.reshape(TBLK, 8, BTIL, 128)
        .transpose(0, 2, 1, 3)
        .reshape(n_total)
    )
    flat_table = jnp.pad(
        table, ((0, TBL_PAD - table.shape[0]), (0, 0))).reshape(TBL_PAD)

    mesh = plsc.VectorSubcoreMesh(core_axis_name="c", subcore_axis_name="s")
    flat_out = pl.kernel(
        _gather_body,
        out_type=jax.ShapeDtypeStruct((n_total,), jnp.float32),
        mesh=mesh,
        scratch_types=(
            [pltpu.VMEM((CHUNK,), jnp.int32)] * 8
            + [pltpu.VMEM((CHUNK,), jnp.float32)] * 4
            + [pltpu.MemorySpace.VMEM_SHARED((TBL_PAD,), jnp.float32)]
            + [pltpu.VMEM((TBL_BOUNCE,), jnp.float32)]
            + [pltpu.SemaphoreType.DMA] * 13
        ),
    )(flat_idx, flat_table)
    return flat_out.reshape(T_DIM, B_DIM, 1).transpose(1, 0, 2)
